# Initial kernel scaffold; baseline (speedup 1.0000x reference)
import jax
import jax.numpy as jnp
from jax.experimental import pallas as pl


def kernel(pe, position_ids, length):
    raise NotImplementedError("write your pallas kernel here")


# speedup vs baseline: 1.6377x; 1.6377x over previous
import functools

import jax
import jax.numpy as jnp
from jax import lax
from jax.experimental import pallas as pl
from jax.experimental.pallas import tpu as pltpu
from jax.experimental.pallas import tpu_sc as plsc

_NC = 2
_NS = 16
_NW = _NC * _NS
_CHUNK = 16


def kernel(pe, position_ids, length):
    num_indices = position_ids.shape[1]
    width = pe.shape[1]
    rows_per_w = num_indices // _NW
    nchunks = rows_per_w // _CHUNK
    idx = position_ids.reshape(num_indices).astype(jnp.int32)
    mesh = plsc.VectorSubcoreMesh(core_axis_name="core", subcore_axis_name="subcore")

    @functools.partial(
        pl.kernel,
        out_type=jax.ShapeDtypeStruct((num_indices, width), pe.dtype),
        mesh=mesh,
        scratch_types=[
            pltpu.VMEM((rows_per_w,), jnp.int32),
            pltpu.VMEM((_CHUNK, width), pe.dtype),
            pltpu.VMEM((_CHUNK, width), pe.dtype),
            pltpu.SemaphoreType.DMA,
            pltpu.SemaphoreType.DMA,
            pltpu.SemaphoreType.DMA,
            pltpu.SemaphoreType.DMA,
        ],
    )
    def gather_kernel(x_hbm, i_hbm, o_hbm, idx_v, buf0, buf1,
                      gsem0, gsem1, wsem0, wsem1):
        wid = lax.axis_index("subcore") * _NC + lax.axis_index("core")
        base = wid * rows_per_w
        pltpu.sync_copy(i_hbm.at[pl.ds(base, rows_per_w)], idx_v)

        bufs = (buf0, buf1)
        gsems = (gsem0, gsem1)
        wsems = (wsem0, wsem1)
        gathers = [None] * nchunks
        writes = [None] * nchunks
        for c in range(nchunks):
            b = c % 2
            if c >= 2:
                writes[c - 2].wait()
            gathers[c] = pltpu.async_copy(
                x_hbm.at[idx_v.at[pl.ds(c * _CHUNK, _CHUNK)]], bufs[b], gsems[b])
            if c >= 1:
                gathers[c - 1].wait()
                writes[c - 1] = pltpu.async_copy(
                    bufs[(c - 1) % 2],
                    o_hbm.at[pl.ds(base + (c - 1) * _CHUNK, _CHUNK)],
                    wsems[(c - 1) % 2])
        last = nchunks - 1
        gathers[last].wait()
        writes[last] = pltpu.async_copy(
            bufs[last % 2], o_hbm.at[pl.ds(base + last * _CHUNK, _CHUNK)],
            wsems[last % 2])
        writes[last - 1].wait()
        writes[last].wait()

    out = gather_kernel(pe, idx)
    return out[None]
